# baseline (speedup 1.0000x reference)
import functools

import jax
import jax.numpy as jnp
from jax import lax
from jax.experimental import pallas as pl
from jax.experimental.pallas import tpu as pltpu
from jax.experimental.pallas import tpu_sc as plsc

_N = 320000
_D = 128
_S = 256
_NC = 2
_NS = 16
_NW = _NC * _NS
_RPT = _N // _NW
_CH = 400
_NCHUNK = _RPT // _CH
_G = 16
_ABLK = 16000


def _tc_attn(x, Wp, b2):
    def body(x_ref, w_ref, b_ref, o_ref):
        z = jnp.dot(x_ref[...], w_ref[...],
                    preferred_element_type=jnp.float32)
        o_ref[...] = jax.nn.sigmoid(z[:, 0:1] + b_ref[0, 0])

    return pl.pallas_call(
        body,
        grid=(_N // _ABLK,),
        in_specs=[
            pl.BlockSpec((_ABLK, _D), lambda i: (i, 0)),
            pl.BlockSpec((_D, _D), lambda i: (0, 0)),
            pl.BlockSpec(memory_space=pltpu.SMEM),
        ],
        out_specs=pl.BlockSpec((_ABLK, 1), lambda i: (i, 0)),
        out_shape=jax.ShapeDtypeStruct((_N, 1), jnp.float32),
    )(x, Wp, b2)


def _sc_partials(x, seg, attn):
    mesh = plsc.VectorSubcoreMesh(core_axis_name="c", subcore_axis_name="s")

    @functools.partial(
        pl.kernel,
        mesh=mesh,
        out_type=jax.ShapeDtypeStruct((_NW, _S, _D), jnp.float32),
        compiler_params=pltpu.CompilerParams(needs_layout_passes=False),
        scratch_types=[
            pltpu.VMEM((_CH, _D), jnp.float32),
            pltpu.VMEM((_CH + 16,), jnp.int32),
            pltpu.VMEM((_CH + 16,), jnp.float32),
            pltpu.VMEM((_S, _D), jnp.float32),
        ],
    )
    def k(x_hbm, seg_hbm, attn_hbm, part_hbm, xbuf, sbuf, abuf, acc):
        cid = lax.axis_index("c")
        sid = lax.axis_index("s")
        wid = sid * _NC + cid
        base = wid * _RPT

        zeros = jnp.zeros((16,), jnp.float32)

        @plsc.parallel_loop(0, _S)
        def zero_body(i):
            for k2 in range(8):
                acc[i, pl.ds(16 * k2, 16)] = zeros

        _dn = lax.GatherDimensionNumbers(
            offset_dims=(), collapsed_slice_dims=(0,), start_index_map=(0,))

        def _bcast(v, j):
            idx = jnp.full((16,), j, jnp.int32)
            return lax.gather(
                v, idx[:, None], _dn, slice_sizes=(1,),
                mode=lax.GatherScatterMode.PROMISE_IN_BOUNDS)

        def chunk_body(ci, _):
            start = base + ci * _CH
            pltpu.sync_copy(x_hbm.at[pl.ds(start, _CH)], xbuf)
            pltpu.sync_copy(seg_hbm.at[pl.ds(start, _CH)],
                            sbuf.at[pl.ds(0, _CH)])
            pltpu.sync_copy(attn_hbm.at[pl.ds(start, _CH)],
                            abuf.at[pl.ds(0, _CH)])

            @plsc.parallel_loop(0, _CH // _G, 1, unroll=4)
            def row_group(g):
                segv = sbuf[pl.ds(g * _G, 16)]
                av = abuf[pl.ds(g * _G, 16)]
                for j in range(_G):
                    r = g * _G + j
                    avj = _bcast(av, j)
                    s0 = segv[j]
                    for k2 in range(8):
                        plsc.addupdate(acc.at[s0, pl.ds(16 * k2, 16)],
                                       xbuf[r, pl.ds(16 * k2, 16)] * avj)
            return 0

        lax.fori_loop(0, _NCHUNK, chunk_body, 0)

        pltpu.sync_copy(acc, part_hbm.at[wid])

    return k(x, seg, attn)


def _combine(parts):
    def body(p_ref, o_ref):
        o_ref[...] = jnp.sum(p_ref[...], axis=0)

    return pl.pallas_call(
        body,
        out_shape=jax.ShapeDtypeStruct((_S, _D), jnp.float32),
    )(parts)


def kernel(x, batch, W, b):
    seg = batch.astype(jnp.int32)
    b2 = b.astype(jnp.float32).reshape(1, 1)
    Wp = jnp.pad(W.astype(jnp.float32), ((0, 0), (0, _D - 1)))
    attn = _tc_attn(x, Wp, b2)
    parts = _sc_partials(x, seg, attn.reshape(-1))
    return _combine(parts)


# speedup vs baseline: 1.4808x; 1.4808x over previous
import functools

import jax
import jax.numpy as jnp
from jax import lax
from jax.experimental import pallas as pl
from jax.experimental.pallas import tpu as pltpu
from jax.experimental.pallas import tpu_sc as plsc

_N = 320000
_D = 128
_S = 256
_NC = 2
_NS = 16
_NW = _NC * _NS
_RPT = _N // _NW
_CH = 400
_NCHUNK = _RPT // _CH
_G = 16
_ABLK = 8000


def _tc_attn(x, Wp, b2):
    def body(x_ref, w_ref, b_ref, o_ref):
        z = jnp.dot(x_ref[...], w_ref[...],
                    preferred_element_type=jnp.float32)
        o_ref[...] = jax.nn.sigmoid(z[:, 0:1] + b_ref[0, 0])

    return pl.pallas_call(
        body,
        grid=(_N // _ABLK,),
        in_specs=[
            pl.BlockSpec((_ABLK, _D), lambda i: (i, 0)),
            pl.BlockSpec((_D, _D), lambda i: (0, 0)),
            pl.BlockSpec(memory_space=pltpu.SMEM),
        ],
        out_specs=pl.BlockSpec((_ABLK, 1), lambda i: (i, 0)),
        out_shape=jax.ShapeDtypeStruct((_N, 1), jnp.float32),
    )(x, Wp, b2)


def _sc_partials(x, seg, attn):
    mesh = plsc.VectorSubcoreMesh(core_axis_name="c", subcore_axis_name="s")

    @functools.partial(
        pl.kernel,
        mesh=mesh,
        out_type=jax.ShapeDtypeStruct((_NW, _S, _D), jnp.float32),
        compiler_params=pltpu.CompilerParams(needs_layout_passes=False),
        scratch_types=[
            pltpu.VMEM((_CH, _D), jnp.float32),
            pltpu.VMEM((_CH + 16,), jnp.int32),
            pltpu.VMEM((_CH + 16,), jnp.float32),
            pltpu.VMEM((_S, _D), jnp.float32),
        ],
    )
    def k(x_hbm, seg_hbm, attn_hbm, part_hbm, xbuf, sbuf, abuf, acc):
        cid = lax.axis_index("c")
        sid = lax.axis_index("s")
        wid = sid * _NC + cid
        base = wid * _RPT

        zeros = jnp.zeros((16,), jnp.float32)

        @plsc.parallel_loop(0, _S)
        def zero_body(i):
            for k2 in range(8):
                acc[i, pl.ds(16 * k2, 16)] = zeros

        _dn = lax.GatherDimensionNumbers(
            offset_dims=(), collapsed_slice_dims=(0,), start_index_map=(0,))

        def _bcast(v, j):
            idx = jnp.full((16,), j, jnp.int32)
            return lax.gather(
                v, idx[:, None], _dn, slice_sizes=(1,),
                mode=lax.GatherScatterMode.PROMISE_IN_BOUNDS)

        def chunk_body(ci, _):
            start = base + ci * _CH
            pltpu.sync_copy(x_hbm.at[pl.ds(start, _CH)], xbuf)
            pltpu.sync_copy(seg_hbm.at[pl.ds(start, _CH)],
                            sbuf.at[pl.ds(0, _CH)])
            pltpu.sync_copy(attn_hbm.at[pl.ds(start, _CH)],
                            abuf.at[pl.ds(0, _CH)])

            @plsc.parallel_loop(0, _CH // _G, 1, unroll=2)
            def row_group(g):
                segv = sbuf[pl.ds(g * _G, 16)]
                av = abuf[pl.ds(g * _G, 16)]
                for j in range(_G):
                    r = g * _G + j
                    avj = _bcast(av, j)
                    s0 = segv[j]
                    for k2 in range(8):
                        plsc.addupdate(acc.at[s0, pl.ds(16 * k2, 16)],
                                       xbuf[r, pl.ds(16 * k2, 16)] * avj)
            return 0

        lax.fori_loop(0, _NCHUNK, chunk_body, 0)

        pltpu.sync_copy(acc, part_hbm.at[wid])

    return k(x, seg, attn)


def _combine(parts):
    def body(p_ref, o_ref):
        o_ref[...] = jnp.sum(p_ref[...], axis=0)

    return pl.pallas_call(
        body,
        out_shape=jax.ShapeDtypeStruct((_S, _D), jnp.float32),
    )(parts)


def kernel(x, batch, W, b):
    seg = batch.astype(jnp.int32)
    b2 = b.astype(jnp.float32).reshape(1, 1)
    Wp = jnp.pad(W.astype(jnp.float32), ((0, 0), (0, _D - 1)))
    attn = _tc_attn(x, Wp, b2)
    parts = _sc_partials(x, seg, attn.reshape(-1))
    return _combine(parts)
